# baseline (speedup 1.0000x reference)
import functools

import jax
import jax.numpy as jnp
from jax import lax
from jax.experimental import pallas as pl
from jax.experimental.pallas import tpu as pltpu
from jax.experimental.pallas import tpu_sc as plsc

_B, _S, _F, _D = 1024, 63, 26, 128
_NC, _NS = 2, 16
_NW = _NC * _NS
_BPW = _B // _NW


def _sc_lut_body(segs, x2_ref, wflat_ref, out_ref, xv, wv, rv, outv):
    wid = lax.axis_index("s") * _NC + lax.axis_index("c")
    pltpu.sync_copy(wflat_ref, wv)
    iota = lax.iota(jnp.int32, 16)

    def batch_body(bi, carry):
        b = wid * _BPW + bi
        pltpu.sync_copy(x2_ref.at[b], xv)
        for g in range(4):
            for k, (col, off) in enumerate(segs):
                idxv = iota * _F + (g * 16 * _F + col)
                colval = plsc.load_gather(xv, [idxv])
                r = (colval.astype(jnp.int32) + off) * _D
                rv[pl.ds(k * 64 + g * 16, 16)] = r

        def tok_body(t, c2):
            racc = []
            for k in range(5):
                bvec = lax.broadcast(t + k * 64, (16,))
                racc.append(plsc.load_gather(rv, [bvec]))
            for v in range(8):
                lane = iota + v * 16
                acc = plsc.load_gather(wv, [racc[0] + lane])
                for k in range(1, 5):
                    acc = acc + plsc.load_gather(wv, [racc[k] + lane])
                outv[t, pl.ds(v * 16, 16)] = acc
            return c2

        lax.fori_loop(0, _S, tok_body, 0)
        pltpu.sync_copy(outv, out_ref.at[b])
        return carry

    lax.fori_loop(0, _BPW, batch_body, 0)


def _tc_combine_kernel(x_ref, card_ref, lut_ref, w26_ref, out_ref, *, bb):
    w26 = w26_ref[...]
    for i in range(bb):
        coin = jnp.dot(x_ref[i], w26, preferred_element_type=jnp.float32)
        out_ref[i] = coin + card_ref[i] + lut_ref[i, :_S, :]


def kernel(x, card_emb_out, turn_table, pos_table, civ_table, face_table, action_table, coin_W, coin_b):
    B, S, F = x.shape
    D = card_emb_out.shape[-1]

    n = (S - 6) // 19
    lookup = {3: 0, 4: 4, 5: 9, 6: 15, 7: 22}
    o = lookup.get(n, -100)

    w_lut = jnp.concatenate(
        [turn_table, pos_table, civ_table, face_table, action_table], axis=0)
    w_lut = w_lut.at[:20].add(coin_b[None, :])
    wflat = w_lut.reshape(65 * D)
    segs = ((0, 0), (3, 20 + o), (4, 50), (5, 58), (2, 61))

    x2 = jnp.pad(x.reshape(B, S * F), ((0, 0), (0, 1664 - S * F)))

    sc_lut = pl.kernel(
        functools.partial(_sc_lut_body, segs),
        out_type=jax.ShapeDtypeStruct((B, 64, D), jnp.float32),
        mesh=plsc.VectorSubcoreMesh(core_axis_name="c", subcore_axis_name="s",
                                    num_cores=_NC, num_subcores=_NS),
        compiler_params=pltpu.CompilerParams(needs_layout_passes=False),
        scratch_types=[
            pltpu.VMEM((1664,), jnp.float32),
            pltpu.VMEM((65 * _D,), jnp.float32),
            pltpu.VMEM((5 * 64,), jnp.int32),
            pltpu.VMEM((64, _D), jnp.float32),
        ],
    )
    lut = sc_lut(x2, wflat)

    w26 = jnp.zeros((F, D), dtype=jnp.float32).at[6:].set(coin_W)
    bb = 128
    grid = B // bb
    return pl.pallas_call(
        functools.partial(_tc_combine_kernel, bb=bb),
        grid=(grid,),
        compiler_params=pltpu.CompilerParams(dimension_semantics=("parallel",)),
        in_specs=[
            pl.BlockSpec((bb, S, F), lambda i: (i, 0, 0)),
            pl.BlockSpec((bb, S, D), lambda i: (i, 0, 0)),
            pl.BlockSpec((bb, 64, D), lambda i: (i, 0, 0)),
            pl.BlockSpec(w26.shape, lambda i: (0, 0)),
        ],
        out_specs=pl.BlockSpec((bb, S, D), lambda i: (i, 0, 0)),
        out_shape=jax.ShapeDtypeStruct((B, S, D), jnp.float32),
    )(x, card_emb_out, lut, w26)
